# baseline (speedup 1.0000x reference)
import functools

import jax
import jax.numpy as jnp
from jax import lax
from jax.experimental import pallas as pl
from jax.experimental.pallas import tpu as pltpu
from jax.experimental.pallas import tpu_sc as plsc

NN = 10000
EE = 640000
DD = 128
HH = 16
NC = 2
NS = 16
CHUNK = 400
EPT = EE // NS
NCH = EPT // CHUNK
NN_PAD = 10240
RPT = NN_PAD // NS

_sc_mesh = plsc.VectorSubcoreMesh(
    core_axis_name="c", subcore_axis_name="s", num_cores=NC, num_subcores=NS
)


@functools.partial(
    pl.kernel,
    out_type=jax.ShapeDtypeStruct((NN_PAD,), jnp.float32),
    mesh=_sc_mesh,
    scratch_types=[
        pltpu.VMEM((NCH, CHUNK), jnp.int32),
        pltpu.VMEM((CHUNK,), jnp.float32),
        pltpu.VMEM((RPT,), jnp.float32),
        pltpu.VMEM_SHARED((NN_PAD,), jnp.float32),
        pltpu.SemaphoreType.DMA,
        pltpu.SemaphoreType.DMA,
    ],
    compiler_params=pltpu.CompilerParams(use_tc_tiling_on_sc=False, needs_layout_passes=False),
)
def _sc_degree(dst_hbm, dis_out, dst_v, ones_v, dv_v, acc, gsem, ssem):
    cid = lax.axis_index("c")
    sid = lax.axis_index("s")
    stripe = pl.ds(sid * RPT, RPT)

    idx_c = pltpu.async_copy(dst_hbm.at[sid], dst_v, gsem)

    def fill_zero(j, c):
        dv_v[pl.ds(j * 16, 16)] = jnp.zeros((16,), jnp.float32)
        return c

    lax.fori_loop(0, RPT // 16, fill_zero, 0)

    def fill_one(j, c):
        ones_v[pl.ds(j * 16, 16)] = jnp.ones((16,), jnp.float32)
        return c

    lax.fori_loop(0, CHUNK // 16, fill_one, 0)

    pltpu.sync_copy(dv_v, acc.at[stripe])
    idx_c.wait()
    plsc.subcore_barrier()

    K = 10
    for g in range(0, NCH, K):
        descs = [
            pltpu.async_copy(ones_v, acc.at[dst_v.at[g + j]], ssem, add=True)
            for j in range(K)
        ]
        for d in descs:
            d.wait()
    plsc.subcore_barrier()

    pltpu.sync_copy(acc.at[stripe], dv_v)

    def newton(j, c):
        sl = pl.ds(j * 16, 16)
        d = dv_v[sl] + 1.0
        ib = lax.bitcast_convert_type(d, jnp.int32)
        y = lax.bitcast_convert_type(
            jnp.int32(0x5F3759DF) - (ib >> 1), jnp.float32
        )
        for _ in range(3):
            y = y * (1.5 - 0.5 * d * y * y)
        dv_v[sl] = y
        return c

    lax.fori_loop(0, RPT // 16, newton, 0)

    @pl.when(sid // (NS // NC) == cid)
    def _():
        pltpu.sync_copy(dv_v, dis_out.at[stripe])


@functools.partial(
    pl.kernel,
    out_type=(
        jax.ShapeDtypeStruct((NN_PAD,), jnp.float32),
        jax.ShapeDtypeStruct((NC, NN_PAD, HH), jnp.float32),
    ),
    mesh=_sc_mesh,
    scratch_types=[
        pltpu.VMEM((NCH, CHUNK), jnp.int32),
        pltpu.VMEM((NCH, CHUNK), jnp.int32),
        pltpu.VMEM((2, CHUNK, HH), jnp.float32),
        pltpu.VMEM((RPT, HH), jnp.float32),
        pltpu.VMEM((RPT, HH), jnp.float32),
        pltpu.VMEM((RPT,), jnp.float32),
        pltpu.VMEM((3, HH, HH), jnp.float32),
        pltpu.VMEM((6, HH), jnp.float32),
        pltpu.VMEM((RPT,), jnp.float32),
        pltpu.VMEM_SHARED((NN_PAD, HH), jnp.float32),
        pltpu.SemaphoreType.DMA,
        pltpu.SemaphoreType.DMA,
    ],
    compiler_params=pltpu.CompilerParams(use_tc_tiling_on_sc=False, needs_layout_passes=False),
)
def _sc_mega(g1_hbm, src_hbm, dst_hbm, dis_hbm, wall_hbm, ball_hbm,
             out_z, gcopy, src_v, dst_v, rows_v, stage_v, gcur_v, dis_v,
             w_v, ball_v, zbuf_v, acc, gsem, ssem):
    cid = lax.axis_index("c")
    sid = lax.axis_index("s")
    stripe = pl.ds(sid * RPT, RPT)
    iota16 = lax.iota(jnp.int32, 16)

    pre = [
        pltpu.async_copy(src_hbm.at[sid], src_v, gsem),
        pltpu.async_copy(dst_hbm.at[sid], dst_v, gsem),
        pltpu.async_copy(dis_hbm.at[stripe], dis_v, gsem),
        pltpu.async_copy(wall_hbm, w_v, gsem),
        pltpu.async_copy(ball_hbm, ball_v, gsem),
        pltpu.async_copy(g1_hbm.at[stripe], gcur_v, gsem),
    ]
    for c in pre:
        c.wait()

    for l in range(4):
        def fill_zero(j, c):
            stage_v[j, :] = jnp.zeros((HH,), jnp.float32)
            return c

        lax.fori_loop(0, RPT, fill_zero, 0)
        pltpu.sync_copy(stage_v, acc.at[stripe])
        plsc.subcore_barrier()

        gsrc = g1_hbm if l == 0 else gcopy.at[cid]

        gat = pltpu.async_copy(gsrc.at[src_v.at[0]], rows_v.at[0], gsem)
        prev = None
        for i in range(NCH):
            cur, nxt = i % 2, (i + 1) % 2
            if prev is not None:
                prev.wait()
            if i + 1 < NCH:
                gat_next = pltpu.async_copy(
                    gsrc.at[src_v.at[i + 1]], rows_v.at[nxt], gsem
                )
            gat.wait()
            prev = pltpu.async_copy(
                rows_v.at[cur], acc.at[dst_v.at[i]], ssem, add=True
            )
            if i + 1 < NCH:
                gat = gat_next
        prev.wait()
        plsc.subcore_barrier()

        pltpu.sync_copy(acc.at[stripe], stage_v)
        b_vec = ball_v[l, :]
        last = l == 3

        def combine(bi, c):
            base = bi * 16
            dis_col = dis_v[pl.ds(base, 16)]
            for j in range(16):
                s = stage_v[base + j, :] + gcur_v[base + j, :]
                xh = s * dis_col[j] + b_vec
                if not last:
                    xh = jnp.maximum(xh, 0.0)
                stage_v[base + j, :] = xh
            return c

        lax.fori_loop(0, RPT // 16, combine, 0)

        if not last:
            def batch(bi, c):
                ridx = bi * 16 + iota16
                dis_col = dis_v[pl.ds(bi * 16, 16)]
                accs = [jnp.zeros((16,), jnp.float32) for _ in range(HH)]
                for k in range(HH):
                    col = plsc.load_gather(
                        stage_v, [ridx, jnp.full((16,), k, jnp.int32)]
                    )
                    wrow = w_v[l, k, :]
                    accs = [a + col * wrow[j] for j, a in enumerate(accs)]
                for j in range(HH):
                    plsc.store_scatter(
                        gcur_v,
                        [ridx, jnp.full((16,), j, jnp.int32)],
                        accs[j] * dis_col,
                    )
                return c

            lax.fori_loop(0, RPT // 16, batch, 0)
            wr = pltpu.async_copy(gcur_v, gcopy.at[cid].at[stripe], gsem)
            wr.wait()
            plsc.subcore_barrier()
        else:
            fcw_row = ball_v[4, :]
            fcb_row = ball_v[5, :]

            def batch4(bi, c):
                ridx = bi * 16 + iota16
                z = jnp.zeros((16,), jnp.float32)
                for k in range(HH):
                    col = plsc.load_gather(
                        stage_v, [ridx, jnp.full((16,), k, jnp.int32)]
                    )
                    z = z + col * fcw_row[k]
                z = z + fcb_row[0]
                zbuf_v[pl.ds(bi * 16, 16)] = 1.0 / (1.0 + jnp.exp(-z))
                return c

            lax.fori_loop(0, RPT // 16, batch4, 0)

            @pl.when(sid // (NS // NC) == cid)
            def _():
                pltpu.sync_copy(zbuf_v, out_z.at[stripe])


def _tc_first_body(disp_ref, x_ref, w_ref, g_ref):
    h = jnp.dot(x_ref[...], w_ref[...], preferred_element_type=jnp.float32)
    g_ref[0:NN, :] = h * disp_ref[...]
    g_ref[NN:NN_PAD, :] = jnp.zeros((NN_PAD - NN, HH), jnp.float32)


_tc_first = pl.pallas_call(
    _tc_first_body,
    out_shape=jax.ShapeDtypeStruct((NN_PAD, HH), jnp.float32),
)


def kernel(x, edge_index, W1, b1, W2, b2, W3, b3, W4, b4, fc_w, fc_b):
    src3 = edge_index[0].reshape(NS, NCH, CHUNK)
    dst3 = edge_index[1].reshape(NS, NCH, CHUNK)
    wall = jnp.stack([W2, W3, W4])
    ball = jnp.stack(
        [b1, b2, b3, b4, fc_w.reshape(HH), jnp.broadcast_to(fc_b, (HH,))]
    )

    disp = _sc_degree(dst3)
    g1 = _tc_first(disp[:NN, None], x, W1)
    out_z, _ = _sc_mega(g1, src3, dst3, disp, wall, ball)
    return out_z[:NN]


# speedup vs baseline: 1.6288x; 1.6288x over previous
import functools

import jax
import jax.numpy as jnp
from jax import lax
from jax.experimental import pallas as pl
from jax.experimental.pallas import tpu as pltpu
from jax.experimental.pallas import tpu_sc as plsc

NN = 10000
EE = 640000
DD = 128
HH = 16
NC = 2
NS = 16
NW = NC * NS
EPT = EE // NW
CHUNK = 2000
NCHUNKS = EPT // CHUNK
NN_PAD = 10240
RPT = NN_PAD // NS

_sc_mesh = plsc.VectorSubcoreMesh(
    core_axis_name="c", subcore_axis_name="s", num_cores=NC, num_subcores=NS
)


@functools.partial(
    pl.kernel,
    out_type=jax.ShapeDtypeStruct((NC, NN_PAD), jnp.float32),
    mesh=_sc_mesh,
    scratch_types=[
        pltpu.VMEM((NCHUNKS, CHUNK), jnp.int32),
        pltpu.VMEM((CHUNK,), jnp.float32),
        pltpu.VMEM((RPT,), jnp.float32),
        pltpu.VMEM_SHARED((NN_PAD,), jnp.float32),
    ],
    compiler_params=pltpu.CompilerParams(use_tc_tiling_on_sc=False),
)
def _sc_degree(dst_hbm, deg_out, dst_v, ones_v, stage_v, acc):
    cid = lax.axis_index("c")
    sid = lax.axis_index("s")
    wid = cid * NS + sid

    def fill_zero(j, c):
        stage_v[pl.ds(j * 16, 16)] = jnp.zeros((16,), jnp.float32)
        return c

    lax.fori_loop(0, RPT // 16, fill_zero, 0)

    def fill_one(j, c):
        ones_v[pl.ds(j * 16, 16)] = jnp.ones((16,), jnp.float32)
        return c

    lax.fori_loop(0, CHUNK // 16, fill_one, 0)

    pltpu.sync_copy(stage_v, acc.at[pl.ds(sid * RPT, RPT)])
    pltpu.sync_copy(dst_hbm.at[wid], dst_v)
    plsc.subcore_barrier()

    def chunk_body(i, c):
        pltpu.sync_copy(ones_v, acc.at[dst_v.at[i]], add=True)
        return c

    lax.fori_loop(0, NCHUNKS, chunk_body, 0)
    plsc.subcore_barrier()

    pltpu.sync_copy(acc.at[pl.ds(sid * RPT, RPT)], stage_v)
    pltpu.sync_copy(stage_v, deg_out.at[cid].at[pl.ds(sid * RPT, RPT)])


@functools.partial(
    pl.kernel,
    out_type=jax.ShapeDtypeStruct((NC, NN_PAD, HH), jnp.float32),
    mesh=_sc_mesh,
    scratch_types=[
        pltpu.VMEM((NCHUNKS, CHUNK), jnp.int32),
        pltpu.VMEM((NCHUNKS, CHUNK), jnp.int32),
        pltpu.VMEM((2, CHUNK, HH), jnp.float32),
        pltpu.VMEM((RPT, HH), jnp.float32),
        pltpu.VMEM_SHARED((NN_PAD, HH), jnp.float32),
        pltpu.SemaphoreType.DMA,
        pltpu.SemaphoreType.DMA,
    ],
    compiler_params=pltpu.CompilerParams(use_tc_tiling_on_sc=False),
)
def _sc_layer(g_hbm, src_hbm, dst_hbm, agg_out, src_v, dst_v, rows_v, stage_v, acc, gsem, ssem):
    cid = lax.axis_index("c")
    sid = lax.axis_index("s")
    wid = cid * NS + sid

    idx_a = pltpu.async_copy(src_hbm.at[wid], src_v, gsem)
    idx_b = pltpu.async_copy(dst_hbm.at[wid], dst_v, gsem)

    def fill_zero(j, c):
        stage_v[j, :] = jnp.zeros((HH,), jnp.float32)
        return c

    lax.fori_loop(0, RPT, fill_zero, 0)

    pltpu.sync_copy(stage_v, acc.at[pl.ds(sid * RPT, RPT)])
    idx_a.wait()
    idx_b.wait()
    plsc.subcore_barrier()

    gat_cur = pltpu.async_copy(g_hbm.at[src_v.at[0]], rows_v.at[0], gsem)
    prev_scatter = None
    for i in range(NCHUNKS):
        cur, nxt = i % 2, (i + 1) % 2
        if prev_scatter is not None:
            prev_scatter.wait()
        if i + 1 < NCHUNKS:
            gat_next = pltpu.async_copy(
                g_hbm.at[src_v.at[i + 1]], rows_v.at[nxt], gsem
            )
        gat_cur.wait()
        prev_scatter = pltpu.async_copy(
            rows_v.at[cur], acc.at[dst_v.at[i]], ssem, add=True
        )
        if i + 1 < NCHUNKS:
            gat_cur = gat_next
    prev_scatter.wait()
    plsc.subcore_barrier()

    pltpu.sync_copy(acc.at[pl.ds(sid * RPT, RPT)], stage_v)
    pltpu.sync_copy(stage_v, agg_out.at[cid].at[pl.ds(sid * RPT, RPT)])


def _tc_first_body(deg_ref, x_ref, w_ref, dis_ref, g_ref):
    d = deg_ref[0] + deg_ref[1] + 1.0
    dis = lax.rsqrt(d)
    dis_ref[...] = dis
    h = jnp.dot(x_ref[...], w_ref[...], preferred_element_type=jnp.float32)
    g_ref[...] = h * dis


def _tc_mid_body(agg_ref, g_ref, dis_ref, b_ref, w_ref, gout_ref):
    dis = dis_ref[...]
    s = agg_ref[0, :NN, :] + agg_ref[1, :NN, :] + g_ref[...]
    xh = jnp.maximum(s * dis + b_ref[...], 0.0)
    gout_ref[...] = jnp.dot(xh, w_ref[...], preferred_element_type=jnp.float32) * dis


def _tc_final_body(agg_ref, g_ref, dis_ref, b_ref, fcw_ref, fcb_ref, out_ref):
    s = agg_ref[0, :NN, :] + agg_ref[1, :NN, :] + g_ref[...]
    h = s * dis_ref[...] + b_ref[...]
    z = jnp.dot(h, fcw_ref[...], preferred_element_type=jnp.float32) + fcb_ref[...]
    out_ref[...] = jax.nn.sigmoid(z)


_tc_first = pl.pallas_call(
    _tc_first_body,
    out_shape=(
        jax.ShapeDtypeStruct((NN, 1), jnp.float32),
        jax.ShapeDtypeStruct((NN, HH), jnp.float32),
    ),
)

_tc_mid = pl.pallas_call(
    _tc_mid_body,
    out_shape=jax.ShapeDtypeStruct((NN, HH), jnp.float32),
)

_tc_final = pl.pallas_call(
    _tc_final_body,
    out_shape=jax.ShapeDtypeStruct((NN, 1), jnp.float32),
)


def kernel(x, edge_index, W1, b1, W2, b2, W3, b3, W4, b4, fc_w, fc_b):
    src3 = edge_index[0].reshape(NW, NCHUNKS, CHUNK)
    dst3 = edge_index[1].reshape(NW, NCHUNKS, CHUNK)

    deg = _sc_degree(dst3)
    degp = deg[:, :NN, None]
    dis, g = _tc_first(degp, x, W1)

    b1r, b2r, b3r = b1.reshape(1, HH), b2.reshape(1, HH), b3.reshape(1, HH)
    b4r, fcbr = b4.reshape(1, HH), fc_b.reshape(1, 1)

    for b_l, w_next in ((b1r, W2), (b2r, W3), (b3r, W4)):
        agg = _sc_layer(g, src3, dst3)
        g = _tc_mid(agg, g, dis, b_l, w_next)

    agg = _sc_layer(g, src3, dst3)
    out = _tc_final(agg, g, dis, b4r, fc_w, fcbr)
    return out.reshape(-1)
